# baseline (speedup 1.0000x reference)
import numpy as np
import jax
import jax.numpy as jnp
from jax.experimental import pallas as pl
from jax.experimental.pallas import tpu as pltpu

_RESI_NAMES = ['ALA', 'ARG', 'ASN', 'ASP', 'CYS', 'GLN', 'GLU', 'GLY', 'HIS',
               'ILE', 'LEU', 'LYS', 'MET', 'PHE', 'PRO', 'SER', 'THR', 'TRP',
               'TYR', 'VAL']
_NFEA_HASH = {'GLN': 3, 'VAL': 1, 'ASN': 2, 'THR': 1, 'ASP': 2, 'PHE': 2,
              'LEU': 2, 'SER': 1, 'CYS': 1, 'ILE': 1, 'TRP': 2, 'ARG': 5,
              'LYS': 4, 'TYR': 2, 'GLU': 3, 'MET': 3, 'HIS': 2}
_HIDDEN = 64
_NC = 6
_NTYPES = 20
_SCF = 5
_LOG2PI = float(np.log(2.0 * np.pi))

_BLK = 128
_NBLK = 0


def _np_mlp_params(rng, nin, nout):
    w1 = rng.normal(0.0, 0.1, (nin, _HIDDEN)).astype(np.float32)
    w2 = rng.normal(0.0, 0.1, (_HIDDEN, _HIDDEN)).astype(np.float32)
    w3 = rng.normal(0.0, 0.1, (_HIDDEN, nout)).astype(np.float32)
    return {'w1': w1, 'b1': np.zeros((_HIDDEN,), np.float32),
            'w2': w2, 'b2': np.zeros((_HIDDEN,), np.float32),
            'w3': w3, 'b3': np.zeros((nout,), np.float32)}


def _np_make_realnvp(rng, nfea):
    layers = []
    for i in range(_NC):
        if nfea == 1:
            mask = np.zeros((1,), np.float32)
        else:
            mask = np.array([(d + i) % 2 for d in range(nfea)], np.float32)
        layers.append({'mask': mask,
                       's': _np_mlp_params(rng, nfea, nfea),
                       't': _np_mlp_params(rng, nfea, nfea)})
    return layers


def _np_mlp(p, x, final_tanh):
    h = np.tanh(x @ p['w1'] + p['b1'])
    h = np.tanh(h @ p['w2'] + p['b2'])
    o = h @ p['w3'] + p['b3']
    return np.tanh(o) if final_tanh else o


def _build_constants():
    rng = np.random.default_rng(0)
    bb_models, om_models, sc_models = {}, {}, {}
    for i in range(_NTYPES):
        bb_models[i] = _np_make_realnvp(rng, 2)
        om_models[i] = _np_make_realnvp(rng, 1)
        name = _RESI_NAMES[i]
        if name in _NFEA_HASH:
            sc_models[i] = (_np_make_realnvp(rng, _NFEA_HASH[name]),
                            _NFEA_HASH[name])

    H2 = 2 * _HIDDEN
    bb_w1 = np.zeros((_NTYPES, _NC, 2, H2), np.float32)
    bb_b1 = np.zeros((_NTYPES, _NC, H2), np.float32)
    bb_w2 = np.zeros((_NTYPES, _NC, H2, H2), np.float32)
    bb_b2 = np.zeros((_NTYPES, _NC, H2), np.float32)
    bb_w3 = np.zeros((_NTYPES, _NC, H2, 4), np.float32)
    bb_b3 = np.zeros((_NTYPES, _NC, 4), np.float32)
    for j in range(_NTYPES):
        for k in range(_NC):
            layer = bb_models[j][_NC - 1 - k]
            for si, key in enumerate(('s', 't')):
                p = layer[key]
                h0 = si * _HIDDEN
                bb_w1[j, k, :, h0:h0 + _HIDDEN] = p['w1']
                bb_b1[j, k, h0:h0 + _HIDDEN] = p['b1']
                bb_w2[j, k, h0:h0 + _HIDDEN, h0:h0 + _HIDDEN] = p['w2']
                bb_b2[j, k, h0:h0 + _HIDDEN] = p['b2']
                bb_w3[j, k, h0:h0 + _HIDDEN, 2 * si:2 * si + 2] = p['w3']
                bb_b3[j, k, 2 * si:2 * si + 2] = p['b3']
    bb_mask = np.zeros((_NC, 2), np.float32)
    for k in range(_NC):
        i = _NC - 1 - k
        bb_mask[k] = [(0 + i) % 2, (1 + i) % 2]

    sc_w1 = np.zeros((_NTYPES, _NC, _SCF, H2), np.float32)
    sc_b1 = np.zeros((_NTYPES, _NC, H2), np.float32)
    sc_w2 = np.zeros((_NTYPES, _NC, H2, H2), np.float32)
    sc_b2 = np.zeros((_NTYPES, _NC, H2), np.float32)
    sc_w3 = np.zeros((_NTYPES, _NC, H2, 2 * _SCF), np.float32)
    sc_b3 = np.zeros((_NTYPES, _NC, 2 * _SCF), np.float32)
    scmv = np.zeros((_NTYPES, 8, _SCF), np.float32)
    sc_const = np.zeros((_NTYPES,), np.float32)
    for j in range(_NTYPES):
        if j in sc_models:
            layers, n = sc_models[j]
        else:
            layers, n = None, 0
        scmv[j, 6, :n] = 1.0
        sc_const[j] = 0.5 * _LOG2PI * (_SCF - n)
        for k in range(_NC):
            if layers is None:
                scmv[j, k, :] = 1.0
                continue
            layer = layers[_NC - 1 - k]
            scmv[j, k, :n] = layer['mask']
            scmv[j, k, n:] = 1.0
            for si, key in enumerate(('s', 't')):
                p = layer[key]
                h0 = si * _HIDDEN
                f0 = si * _SCF
                sc_w1[j, k, :n, h0:h0 + _HIDDEN] = p['w1']
                sc_b1[j, k, h0:h0 + _HIDDEN] = p['b1']
                sc_w2[j, k, h0:h0 + _HIDDEN, h0:h0 + _HIDDEN] = p['w2']
                sc_b2[j, k, h0:h0 + _HIDDEN] = p['b2']
                sc_w3[j, k, h0:h0 + _HIDDEN, f0:f0 + n] = p['w3']
                sc_b3[j, k, f0:f0 + n] = p['b3']

    om_aff = np.zeros((_NTYPES, 3), np.float32)
    zero = np.zeros((1, 1), np.float32)
    for j in range(_NTYPES):
        a, b, ld = 1.0, 0.0, 0.0
        for layer in reversed(om_models[j]):
            s0 = float(np.tanh(_np_mlp(layer['s'], zero, False))[0, 0])
            t0 = float(_np_mlp(layer['t'], zero, False)[0, 0])
            e = np.exp(-s0)
            a, b = a * e, (b - t0) * e
            ld -= s0
        om_aff[j] = [a, b, ld]

    return dict(bb_w1=bb_w1, bb_b1=bb_b1, bb_w2=bb_w2, bb_b2=bb_b2,
                bb_w3=bb_w3, bb_b3=bb_b3, bb_mask=bb_mask,
                sc_w1=sc_w1, sc_b1=sc_b1, sc_w2=sc_w2, sc_b2=sc_b2,
                sc_w3=sc_w3, sc_b3=sc_b3, scmv=scmv, sc_const=sc_const,
                om_aff=om_aff)


_C = _build_constants()
_BB_MASK = _C['bb_mask']


def _block_body(bt_ref, x_ref, scmv_ref, tab_ref,
                bbw1_ref, bbb1_ref, bbw2_ref, bbb2_ref, bbw3_ref, bbb3_ref,
                scw1_ref, scb1_ref, scw2_ref, scb2_ref, scw3_ref, scb3_ref,
                out_ref):
    x = x_ref[...]
    scmv = scmv_ref[0]
    tab = tab_ref[0, 0]

    z0 = x[:, 0:1]
    z1 = x[:, 1:2]
    xom = x[:, 2]
    valid = scmv[6]
    zsc = x[:, 3:8] * valid[None, :]

    ld_bb = jnp.zeros((x.shape[0], 1), jnp.float32)
    ld_sc = jnp.zeros(x.shape[0], jnp.float32)

    def mlp_cat(zm, w1, b1, w2, b2, w3, b3):
        h = jnp.tanh(jnp.dot(zm, w1, preferred_element_type=jnp.float32)
                     + b1[None, :])
        h = jnp.tanh(jnp.dot(h, w2, preferred_element_type=jnp.float32)
                     + b2[None, :])
        return jnp.dot(h, w3, preferred_element_type=jnp.float32) + b3[None, :]

    for k in range(_NC):
        a = int(_BB_MASK[k][1] > 0.5)
        b = 1 - a
        u = z1 if a == 1 else z0
        zi = z0 if a == 1 else z1
        h = jnp.tanh(u * bbw1_ref[0, k, a][None, :] + bbb1_ref[0, k][None, :])
        h = jnp.tanh(jnp.dot(h, bbw2_ref[0, k],
                             preferred_element_type=jnp.float32)
                     + bbb2_ref[0, k][None, :])
        o = jnp.dot(h, bbw3_ref[0, k],
                    preferred_element_type=jnp.float32) + bbb3_ref[0, k][None, :]
        s = jnp.tanh(o[:, b:b + 1])
        t = o[:, 2 + b:3 + b]
        zi = (zi - t) * jnp.exp(-s)
        if a == 1:
            z0 = zi
        else:
            z1 = zi
        ld_bb = ld_bb - s

        msc = scmv[k]
        zm5 = zsc * msc[None, :]
        o5 = mlp_cat(zm5, scw1_ref[0, k], scb1_ref[0, k], scw2_ref[0, k],
                     scb2_ref[0, k], scw3_ref[0, k], scb3_ref[0, k])
        s5 = jnp.tanh(o5[:, :_SCF]) * (1.0 - msc)[None, :]
        t5 = o5[:, _SCF:] * (1.0 - msc)[None, :]
        zsc = zm5 + (1.0 - msc)[None, :] * (zsc - t5) * jnp.exp(-s5)
        ld_sc = ld_sc - jnp.sum(s5, axis=-1)

    bb_logp = (-0.5 * (z0 * z0 + z1 * z1 + 2.0 * _LOG2PI) + ld_bb)[:, 0]
    sc_logp = (-0.5 * jnp.sum(zsc * zsc, axis=-1) - 0.5 * _LOG2PI * _SCF
               + tab[4] + ld_sc)
    zo = tab[0] * xom + tab[1]
    om_logp = -0.5 * (zo * zo + _LOG2PI) + tab[2]

    bb_p = jnp.minimum(bb_logp * tab[5], 5.0)
    om_p = om_logp * tab[6]
    sc_p = jnp.minimum(sc_logp * tab[3], 5.0)
    vals = jnp.clip(-(bb_p + om_p + sc_p), 0.0, 5.0)
    out_ref[...] = vals.reshape(out_ref.shape)


def kernel(atom_description, angles, alternatives, weightBB, weightOmega,
           weightSC):
    B, Cn, R, A, F = angles.shape
    n_res = B * Cn * R
    apr = atom_description.shape[0] // n_res
    resname = atom_description.reshape(n_res, apr,
                                       atom_description.shape[1])[:, 0, 3]
    resname = resname.astype(jnp.int32)

    nblk = n_res // _BLK + _NTYPES
    npad = nblk * _BLK

    order = jnp.argsort(resname).astype(jnp.int32)
    counts = jnp.bincount(resname, length=_NTYPES).astype(jnp.int32)
    padded = ((counts + _BLK - 1) // _BLK) * _BLK
    pstart = jnp.concatenate([jnp.zeros((1,), jnp.int32),
                              jnp.cumsum(padded)[:-1].astype(jnp.int32)])
    start = jnp.concatenate([jnp.zeros((1,), jnp.int32),
                             jnp.cumsum(counts)[:-1].astype(jnp.int32)])
    rs = resname[order]
    p = jnp.arange(n_res, dtype=jnp.int32)
    slot_sorted = pstart[rs] + (p - start[rs])
    slot_of_res = jnp.zeros((n_res,), jnp.int32).at[order].set(slot_sorted)
    res_of_slot = jnp.zeros((npad,), jnp.int32).at[slot_sorted].set(order)
    bt = jnp.zeros((nblk,), jnp.int32).at[slot_sorted // _BLK].set(rs)

    xg = angles.reshape(n_res, A * F)[res_of_slot].reshape(npad * A, F)

    mult_bb = 1.0 - jnp.tanh(-weightBB[0])
    mult_om = 1.0 - jnp.tanh(-weightOmega[0])
    mult_sc = 1.0 - jnp.tanh(-weightSC)
    tab = jnp.stack([
        jnp.asarray(_C['om_aff'][:, 0]),
        jnp.asarray(_C['om_aff'][:, 1]),
        jnp.asarray(_C['om_aff'][:, 2]),
        mult_sc,
        jnp.asarray(_C['sc_const']),
        jnp.full((_NTYPES,), mult_bb),
        jnp.full((_NTYPES,), mult_om),
        jnp.zeros((_NTYPES,)),
    ], axis=1).astype(jnp.float32).reshape(_NTYPES, 1, 8)

    def im_x(b, bt_r):
        return (b, 0)

    def im_t2(b, bt_r):
        return (bt_r[b], 0)

    def im_t3(b, bt_r):
        return (bt_r[b], 0, 0)

    def im_t4(b, bt_r):
        return (bt_r[b], 0, 0, 0)

    grid_spec = pltpu.PrefetchScalarGridSpec(
        num_scalar_prefetch=1,
        grid=(nblk,),
        in_specs=[
            pl.BlockSpec((_BLK * A, F), im_x),
            pl.BlockSpec((1, 8, _SCF), im_t3),
            pl.BlockSpec((1, 1, 8), im_t3),
            pl.BlockSpec((1, _NC, 2, 2 * _HIDDEN), im_t4),
            pl.BlockSpec((1, _NC, 2 * _HIDDEN), im_t3),
            pl.BlockSpec((1, _NC, 2 * _HIDDEN, 2 * _HIDDEN), im_t4),
            pl.BlockSpec((1, _NC, 2 * _HIDDEN), im_t3),
            pl.BlockSpec((1, _NC, 2 * _HIDDEN, 4), im_t4),
            pl.BlockSpec((1, _NC, 4), im_t3),
            pl.BlockSpec((1, _NC, _SCF, 2 * _HIDDEN), im_t4),
            pl.BlockSpec((1, _NC, 2 * _HIDDEN), im_t3),
            pl.BlockSpec((1, _NC, 2 * _HIDDEN, 2 * _HIDDEN), im_t4),
            pl.BlockSpec((1, _NC, 2 * _HIDDEN), im_t3),
            pl.BlockSpec((1, _NC, 2 * _HIDDEN, 2 * _SCF), im_t4),
            pl.BlockSpec((1, _NC, 2 * _SCF), im_t3),
        ],
        out_specs=pl.BlockSpec((_BLK, A), im_x),
    )

    out = pl.pallas_call(
        _block_body,
        grid_spec=grid_spec,
        out_shape=jax.ShapeDtypeStruct((npad, A), jnp.float32),
    )(bt, xg,
      jnp.asarray(_C['scmv']), tab,
      jnp.asarray(_C['bb_w1']), jnp.asarray(_C['bb_b1']),
      jnp.asarray(_C['bb_w2']), jnp.asarray(_C['bb_b2']),
      jnp.asarray(_C['bb_w3']), jnp.asarray(_C['bb_b3']),
      jnp.asarray(_C['sc_w1']), jnp.asarray(_C['sc_b1']),
      jnp.asarray(_C['sc_w2']), jnp.asarray(_C['sc_b2']),
      jnp.asarray(_C['sc_w3']), jnp.asarray(_C['sc_b3']))

    bbScore = out[slot_of_res].reshape(B, Cn, R, A)
    rotamerViolation = jnp.zeros_like(bbScore)
    return (bbScore, rotamerViolation)


# speedup vs baseline: 1.2712x; 1.2712x over previous
import numpy as np
import jax
import jax.numpy as jnp
from jax.experimental import pallas as pl
from jax.experimental.pallas import tpu as pltpu

_RESI_NAMES = ['ALA', 'ARG', 'ASN', 'ASP', 'CYS', 'GLN', 'GLU', 'GLY', 'HIS',
               'ILE', 'LEU', 'LYS', 'MET', 'PHE', 'PRO', 'SER', 'THR', 'TRP',
               'TYR', 'VAL']
_NFEA_HASH = {'GLN': 3, 'VAL': 1, 'ASN': 2, 'THR': 1, 'ASP': 2, 'PHE': 2,
              'LEU': 2, 'SER': 1, 'CYS': 1, 'ILE': 1, 'TRP': 2, 'ARG': 5,
              'LYS': 4, 'TYR': 2, 'GLU': 3, 'MET': 3, 'HIS': 2}
_HIDDEN = 64
_NC = 6
_NTYPES = 20
_SCF = 5
_LOG2PI = float(np.log(2.0 * np.pi))

_BLK = 128
_NBLK = 0


def _np_mlp_params(rng, nin, nout):
    w1 = rng.normal(0.0, 0.1, (nin, _HIDDEN)).astype(np.float32)
    w2 = rng.normal(0.0, 0.1, (_HIDDEN, _HIDDEN)).astype(np.float32)
    w3 = rng.normal(0.0, 0.1, (_HIDDEN, nout)).astype(np.float32)
    return {'w1': w1, 'b1': np.zeros((_HIDDEN,), np.float32),
            'w2': w2, 'b2': np.zeros((_HIDDEN,), np.float32),
            'w3': w3, 'b3': np.zeros((nout,), np.float32)}


def _np_make_realnvp(rng, nfea):
    layers = []
    for i in range(_NC):
        if nfea == 1:
            mask = np.zeros((1,), np.float32)
        else:
            mask = np.array([(d + i) % 2 for d in range(nfea)], np.float32)
        layers.append({'mask': mask,
                       's': _np_mlp_params(rng, nfea, nfea),
                       't': _np_mlp_params(rng, nfea, nfea)})
    return layers


def _np_mlp(p, x, final_tanh):
    h = np.tanh(x @ p['w1'] + p['b1'])
    h = np.tanh(h @ p['w2'] + p['b2'])
    o = h @ p['w3'] + p['b3']
    return np.tanh(o) if final_tanh else o


def _build_constants():
    rng = np.random.default_rng(0)
    bb_models, om_models, sc_models = {}, {}, {}
    for i in range(_NTYPES):
        bb_models[i] = _np_make_realnvp(rng, 2)
        om_models[i] = _np_make_realnvp(rng, 1)
        name = _RESI_NAMES[i]
        if name in _NFEA_HASH:
            sc_models[i] = (_np_make_realnvp(rng, _NFEA_HASH[name]),
                            _NFEA_HASH[name])

    bb_w1 = np.zeros((_NTYPES, 2 * _NC, 2, _HIDDEN), np.float32)
    bb_b1 = np.zeros((_NTYPES, 2 * _NC, _HIDDEN), np.float32)
    bb_w2 = np.zeros((_NTYPES, 2 * _NC, _HIDDEN, _HIDDEN), np.float32)
    bb_b2 = np.zeros((_NTYPES, 2 * _NC, _HIDDEN), np.float32)
    bb_w3 = np.zeros((_NTYPES, 2 * _NC, _HIDDEN, 2), np.float32)
    bb_b3 = np.zeros((_NTYPES, 2 * _NC, 2), np.float32)
    for j in range(_NTYPES):
        for k in range(_NC):
            layer = bb_models[j][_NC - 1 - k]
            for si, key in enumerate(('s', 't')):
                p = layer[key]
                bb_w1[j, 2 * k + si] = p['w1']
                bb_b1[j, 2 * k + si] = p['b1']
                bb_w2[j, 2 * k + si] = p['w2']
                bb_b2[j, 2 * k + si] = p['b2']
                bb_w3[j, 2 * k + si] = p['w3']
                bb_b3[j, 2 * k + si] = p['b3']
    bb_mask = np.zeros((_NC, 2), np.float32)
    for k in range(_NC):
        i = _NC - 1 - k
        bb_mask[k] = [(0 + i) % 2, (1 + i) % 2]

    sc_w1 = np.zeros((_NTYPES, 2 * _NC, _SCF, _HIDDEN), np.float32)
    sc_b1 = np.zeros((_NTYPES, 2 * _NC, _HIDDEN), np.float32)
    sc_w2 = np.zeros((_NTYPES, 2 * _NC, _HIDDEN, _HIDDEN), np.float32)
    sc_b2 = np.zeros((_NTYPES, 2 * _NC, _HIDDEN), np.float32)
    sc_w3 = np.zeros((_NTYPES, 2 * _NC, _HIDDEN, _SCF), np.float32)
    sc_b3 = np.zeros((_NTYPES, 2 * _NC, _SCF), np.float32)
    scmv = np.zeros((_NTYPES, 8, _SCF), np.float32)
    sc_const = np.zeros((_NTYPES,), np.float32)
    for j in range(_NTYPES):
        if j in sc_models:
            layers, n = sc_models[j]
        else:
            layers, n = None, 0
        scmv[j, 6, :n] = 1.0
        sc_const[j] = 0.5 * _LOG2PI * (_SCF - n)
        for k in range(_NC):
            if layers is None:
                scmv[j, k, :] = 1.0
                continue
            layer = layers[_NC - 1 - k]
            scmv[j, k, :n] = layer['mask']
            scmv[j, k, n:] = 1.0
            for si, key in enumerate(('s', 't')):
                p = layer[key]
                sc_w1[j, 2 * k + si, :n, :] = p['w1']
                sc_b1[j, 2 * k + si] = p['b1']
                sc_w2[j, 2 * k + si] = p['w2']
                sc_b2[j, 2 * k + si] = p['b2']
                sc_w3[j, 2 * k + si, :, :n] = p['w3']
                sc_b3[j, 2 * k + si, :n] = p['b3']

    om_aff = np.zeros((_NTYPES, 3), np.float32)
    zero = np.zeros((1, 1), np.float32)
    for j in range(_NTYPES):
        a, b, ld = 1.0, 0.0, 0.0
        for layer in reversed(om_models[j]):
            s0 = float(np.tanh(_np_mlp(layer['s'], zero, False))[0, 0])
            t0 = float(_np_mlp(layer['t'], zero, False)[0, 0])
            e = np.exp(-s0)
            a, b = a * e, (b - t0) * e
            ld -= s0
        om_aff[j] = [a, b, ld]

    return dict(bb_w1=bb_w1, bb_b1=bb_b1, bb_w2=bb_w2, bb_b2=bb_b2,
                bb_w3=bb_w3, bb_b3=bb_b3, bb_mask=bb_mask,
                sc_w1=sc_w1, sc_b1=sc_b1, sc_w2=sc_w2, sc_b2=sc_b2,
                sc_w3=sc_w3, sc_b3=sc_b3, scmv=scmv, sc_const=sc_const,
                om_aff=om_aff)


_C = _build_constants()
_BB_MASK = _C['bb_mask']


def _block_body(bt_ref, x_ref, scmv_ref, tab_ref,
                bbw1_ref, bbb1_ref, bbw2_ref, bbb2_ref, bbw3_ref, bbb3_ref,
                scw1_ref, scb1_ref, scw2_ref, scb2_ref, scw3_ref, scb3_ref,
                out_ref):
    x = x_ref[...]
    scmv = scmv_ref[0]
    tab = tab_ref[0, 0]

    z0 = x[:, 0:1]
    z1 = x[:, 1:2]
    xom = x[:, 2]
    valid = scmv[6]
    zsc = x[:, 3:8] * valid[None, :]

    ld_bb = jnp.zeros((x.shape[0], 1), jnp.float32)
    ld_sc = jnp.zeros(x.shape[0], jnp.float32)

    def mlp(zm, wref, bref1, w2ref, bref2, w3ref, bref3, idx, final_tanh):
        h = jnp.tanh(jnp.dot(zm, wref[0, idx],
                             preferred_element_type=jnp.float32)
                     + bref1[0, idx][None, :])
        h = jnp.tanh(jnp.dot(h, w2ref[0, idx],
                             preferred_element_type=jnp.float32)
                     + bref2[0, idx][None, :])
        o = jnp.dot(h, w3ref[0, idx],
                    preferred_element_type=jnp.float32) + bref3[0, idx][None, :]
        return jnp.tanh(o) if final_tanh else o

    def mlp_bb(u, a, b, idx, final_tanh):
        h = jnp.tanh(u * bbw1_ref[0, idx, a][None, :] + bbb1_ref[0, idx][None, :])
        h = jnp.tanh(jnp.dot(h, bbw2_ref[0, idx],
                             preferred_element_type=jnp.float32)
                     + bbb2_ref[0, idx][None, :])
        o = jnp.dot(h, bbw3_ref[0, idx, :, b:b + 1],
                    preferred_element_type=jnp.float32) + bbb3_ref[0, idx, b]
        return jnp.tanh(o) if final_tanh else o

    for k in range(_NC):
        a = int(_BB_MASK[k][1] > 0.5)
        b = 1 - a
        u = z1 if a == 1 else z0
        zi = z0 if a == 1 else z1
        s = mlp_bb(u, a, b, 2 * k, True)
        t = mlp_bb(u, a, b, 2 * k + 1, False)
        zi = (zi - t) * jnp.exp(-s)
        if a == 1:
            z0 = zi
        else:
            z1 = zi
        ld_bb = ld_bb - s

        msc = scmv[k]
        zm5 = zsc * msc[None, :]
        s5 = mlp(zm5, scw1_ref, scb1_ref, scw2_ref, scb2_ref, scw3_ref,
                 scb3_ref, 2 * k, True) * (1.0 - msc)[None, :]
        t5 = mlp(zm5, scw1_ref, scb1_ref, scw2_ref, scb2_ref, scw3_ref,
                 scb3_ref, 2 * k + 1, False) * (1.0 - msc)[None, :]
        zsc = zm5 + (1.0 - msc)[None, :] * (zsc - t5) * jnp.exp(-s5)
        ld_sc = ld_sc - jnp.sum(s5, axis=-1)

    bb_logp = (-0.5 * (z0 * z0 + z1 * z1 + 2.0 * _LOG2PI) + ld_bb)[:, 0]
    sc_logp = (-0.5 * jnp.sum(zsc * zsc, axis=-1) - 0.5 * _LOG2PI * _SCF
               + tab[4] + ld_sc)
    zo = tab[0] * xom + tab[1]
    om_logp = -0.5 * (zo * zo + _LOG2PI) + tab[2]

    bb_p = jnp.minimum(bb_logp * tab[5], 5.0)
    om_p = om_logp * tab[6]
    sc_p = jnp.minimum(sc_logp * tab[3], 5.0)
    vals = jnp.clip(-(bb_p + om_p + sc_p), 0.0, 5.0)
    out_ref[...] = vals.reshape(out_ref.shape)


def kernel(atom_description, angles, alternatives, weightBB, weightOmega,
           weightSC):
    B, Cn, R, A, F = angles.shape
    n_res = B * Cn * R
    apr = atom_description.shape[0] // n_res
    resname = atom_description.reshape(n_res, apr,
                                       atom_description.shape[1])[:, 0, 3]
    resname = resname.astype(jnp.int32)

    nblk = n_res // _BLK + _NTYPES
    npad = nblk * _BLK

    order = jnp.argsort(resname).astype(jnp.int32)
    counts = jnp.bincount(resname, length=_NTYPES).astype(jnp.int32)
    padded = ((counts + _BLK - 1) // _BLK) * _BLK
    pstart = jnp.concatenate([jnp.zeros((1,), jnp.int32),
                              jnp.cumsum(padded)[:-1].astype(jnp.int32)])
    start = jnp.concatenate([jnp.zeros((1,), jnp.int32),
                             jnp.cumsum(counts)[:-1].astype(jnp.int32)])
    rs = resname[order]
    p = jnp.arange(n_res, dtype=jnp.int32)
    slot_sorted = pstart[rs] + (p - start[rs])
    slot_of_res = jnp.zeros((n_res,), jnp.int32).at[order].set(slot_sorted)
    res_of_slot = jnp.zeros((npad,), jnp.int32).at[slot_sorted].set(order)
    bt = jnp.zeros((nblk,), jnp.int32).at[slot_sorted // _BLK].set(rs)

    xg = angles.reshape(n_res, A * F)[res_of_slot].reshape(npad * A, F)

    mult_bb = 1.0 - jnp.tanh(-weightBB[0])
    mult_om = 1.0 - jnp.tanh(-weightOmega[0])
    mult_sc = 1.0 - jnp.tanh(-weightSC)
    tab = jnp.stack([
        jnp.asarray(_C['om_aff'][:, 0]),
        jnp.asarray(_C['om_aff'][:, 1]),
        jnp.asarray(_C['om_aff'][:, 2]),
        mult_sc,
        jnp.asarray(_C['sc_const']),
        jnp.full((_NTYPES,), mult_bb),
        jnp.full((_NTYPES,), mult_om),
        jnp.zeros((_NTYPES,)),
    ], axis=1).astype(jnp.float32).reshape(_NTYPES, 1, 8)

    def im_x(b, bt_r):
        return (b, 0)

    def im_t2(b, bt_r):
        return (bt_r[b], 0)

    def im_t3(b, bt_r):
        return (bt_r[b], 0, 0)

    def im_t4(b, bt_r):
        return (bt_r[b], 0, 0, 0)

    grid_spec = pltpu.PrefetchScalarGridSpec(
        num_scalar_prefetch=1,
        grid=(nblk,),
        in_specs=[
            pl.BlockSpec((_BLK * A, F), im_x),
            pl.BlockSpec((1, 8, _SCF), im_t3),
            pl.BlockSpec((1, 1, 8), im_t3),
            pl.BlockSpec((1, 2 * _NC, 2, _HIDDEN), im_t4),
            pl.BlockSpec((1, 2 * _NC, _HIDDEN), im_t3),
            pl.BlockSpec((1, 2 * _NC, _HIDDEN, _HIDDEN), im_t4),
            pl.BlockSpec((1, 2 * _NC, _HIDDEN), im_t3),
            pl.BlockSpec((1, 2 * _NC, _HIDDEN, 2), im_t4),
            pl.BlockSpec((1, 2 * _NC, 2), im_t3),
            pl.BlockSpec((1, 2 * _NC, _SCF, _HIDDEN), im_t4),
            pl.BlockSpec((1, 2 * _NC, _HIDDEN), im_t3),
            pl.BlockSpec((1, 2 * _NC, _HIDDEN, _HIDDEN), im_t4),
            pl.BlockSpec((1, 2 * _NC, _HIDDEN), im_t3),
            pl.BlockSpec((1, 2 * _NC, _HIDDEN, _SCF), im_t4),
            pl.BlockSpec((1, 2 * _NC, _SCF), im_t3),
        ],
        out_specs=pl.BlockSpec((_BLK, A), im_x),
    )

    out = pl.pallas_call(
        _block_body,
        grid_spec=grid_spec,
        out_shape=jax.ShapeDtypeStruct((npad, A), jnp.float32),
    )(bt, xg,
      jnp.asarray(_C['scmv']), tab,
      jnp.asarray(_C['bb_w1']), jnp.asarray(_C['bb_b1']),
      jnp.asarray(_C['bb_w2']), jnp.asarray(_C['bb_b2']),
      jnp.asarray(_C['bb_w3']), jnp.asarray(_C['bb_b3']),
      jnp.asarray(_C['sc_w1']), jnp.asarray(_C['sc_b1']),
      jnp.asarray(_C['sc_w2']), jnp.asarray(_C['sc_b2']),
      jnp.asarray(_C['sc_w3']), jnp.asarray(_C['sc_b3']))

    bbScore = out[slot_of_res].reshape(B, Cn, R, A)
    rotamerViolation = jnp.zeros_like(bbScore)
    return (bbScore, rotamerViolation)


# speedup vs baseline: 1.2758x; 1.0036x over previous
import numpy as np
import jax
import jax.numpy as jnp
from jax.experimental import pallas as pl
from jax.experimental.pallas import tpu as pltpu

_RESI_NAMES = ['ALA', 'ARG', 'ASN', 'ASP', 'CYS', 'GLN', 'GLU', 'GLY', 'HIS',
               'ILE', 'LEU', 'LYS', 'MET', 'PHE', 'PRO', 'SER', 'THR', 'TRP',
               'TYR', 'VAL']
_NFEA_HASH = {'GLN': 3, 'VAL': 1, 'ASN': 2, 'THR': 1, 'ASP': 2, 'PHE': 2,
              'LEU': 2, 'SER': 1, 'CYS': 1, 'ILE': 1, 'TRP': 2, 'ARG': 5,
              'LYS': 4, 'TYR': 2, 'GLU': 3, 'MET': 3, 'HIS': 2}
_HIDDEN = 64
_NC = 6
_NTYPES = 20
_SCF = 5
_LOG2PI = float(np.log(2.0 * np.pi))

_BLK = 128
_NBLK = 0


def _np_mlp_params(rng, nin, nout):
    w1 = rng.normal(0.0, 0.1, (nin, _HIDDEN)).astype(np.float32)
    w2 = rng.normal(0.0, 0.1, (_HIDDEN, _HIDDEN)).astype(np.float32)
    w3 = rng.normal(0.0, 0.1, (_HIDDEN, nout)).astype(np.float32)
    return {'w1': w1, 'b1': np.zeros((_HIDDEN,), np.float32),
            'w2': w2, 'b2': np.zeros((_HIDDEN,), np.float32),
            'w3': w3, 'b3': np.zeros((nout,), np.float32)}


def _np_make_realnvp(rng, nfea):
    layers = []
    for i in range(_NC):
        if nfea == 1:
            mask = np.zeros((1,), np.float32)
        else:
            mask = np.array([(d + i) % 2 for d in range(nfea)], np.float32)
        layers.append({'mask': mask,
                       's': _np_mlp_params(rng, nfea, nfea),
                       't': _np_mlp_params(rng, nfea, nfea)})
    return layers


def _np_mlp(p, x, final_tanh):
    h = np.tanh(x @ p['w1'] + p['b1'])
    h = np.tanh(h @ p['w2'] + p['b2'])
    o = h @ p['w3'] + p['b3']
    return np.tanh(o) if final_tanh else o


def _build_constants():
    rng = np.random.default_rng(0)
    bb_models, om_models, sc_models = {}, {}, {}
    for i in range(_NTYPES):
        bb_models[i] = _np_make_realnvp(rng, 2)
        om_models[i] = _np_make_realnvp(rng, 1)
        name = _RESI_NAMES[i]
        if name in _NFEA_HASH:
            sc_models[i] = (_np_make_realnvp(rng, _NFEA_HASH[name]),
                            _NFEA_HASH[name])

    bb_w1 = np.zeros((_NTYPES, 2 * _NC, 2, _HIDDEN), np.float32)
    bb_b1 = np.zeros((_NTYPES, 2 * _NC, _HIDDEN), np.float32)
    bb_w2 = np.zeros((_NTYPES, 2 * _NC, _HIDDEN, _HIDDEN), np.float32)
    bb_b2 = np.zeros((_NTYPES, 2 * _NC, _HIDDEN), np.float32)
    bb_w3 = np.zeros((_NTYPES, 2 * _NC, _HIDDEN, 2), np.float32)
    bb_b3 = np.zeros((_NTYPES, 2 * _NC, 2), np.float32)
    for j in range(_NTYPES):
        for k in range(_NC):
            layer = bb_models[j][_NC - 1 - k]
            for si, key in enumerate(('s', 't')):
                p = layer[key]
                bb_w1[j, 2 * k + si] = p['w1']
                bb_b1[j, 2 * k + si] = p['b1']
                bb_w2[j, 2 * k + si] = p['w2']
                bb_b2[j, 2 * k + si] = p['b2']
                bb_w3[j, 2 * k + si] = p['w3']
                bb_b3[j, 2 * k + si] = p['b3']
    bb_mask = np.zeros((_NC, 2), np.float32)
    for k in range(_NC):
        i = _NC - 1 - k
        bb_mask[k] = [(0 + i) % 2, (1 + i) % 2]

    sc_w1 = np.zeros((_NTYPES, 2 * _NC, _SCF, _HIDDEN), np.float32)
    sc_b1 = np.zeros((_NTYPES, 2 * _NC, _HIDDEN), np.float32)
    sc_w2 = np.zeros((_NTYPES, 2 * _NC, _HIDDEN, _HIDDEN), np.float32)
    sc_b2 = np.zeros((_NTYPES, 2 * _NC, _HIDDEN), np.float32)
    sc_w3 = np.zeros((_NTYPES, 2 * _NC, _HIDDEN, _SCF), np.float32)
    sc_b3 = np.zeros((_NTYPES, 2 * _NC, _SCF), np.float32)
    scmv = np.zeros((_NTYPES, 8, _SCF), np.float32)
    sc_const = np.zeros((_NTYPES,), np.float32)
    for j in range(_NTYPES):
        if j in sc_models:
            layers, n = sc_models[j]
        else:
            layers, n = None, 0
        scmv[j, 6, :n] = 1.0
        sc_const[j] = 0.5 * _LOG2PI * (_SCF - n)
        for k in range(_NC):
            if layers is None:
                scmv[j, k, :] = 1.0
                continue
            layer = layers[_NC - 1 - k]
            scmv[j, k, :n] = layer['mask']
            scmv[j, k, n:] = 1.0
            for si, key in enumerate(('s', 't')):
                p = layer[key]
                sc_w1[j, 2 * k + si, :n, :] = p['w1']
                sc_b1[j, 2 * k + si] = p['b1']
                sc_w2[j, 2 * k + si] = p['w2']
                sc_b2[j, 2 * k + si] = p['b2']
                sc_w3[j, 2 * k + si, :, :n] = p['w3']
                sc_b3[j, 2 * k + si, :n] = p['b3']

    om_aff = np.zeros((_NTYPES, 3), np.float32)
    zero = np.zeros((1, 1), np.float32)
    for j in range(_NTYPES):
        a, b, ld = 1.0, 0.0, 0.0
        for layer in reversed(om_models[j]):
            s0 = float(np.tanh(_np_mlp(layer['s'], zero, False))[0, 0])
            t0 = float(_np_mlp(layer['t'], zero, False)[0, 0])
            e = np.exp(-s0)
            a, b = a * e, (b - t0) * e
            ld -= s0
        om_aff[j] = [a, b, ld]

    return dict(bb_w1=bb_w1, bb_b1=bb_b1, bb_w2=bb_w2, bb_b2=bb_b2,
                bb_w3=bb_w3, bb_b3=bb_b3, bb_mask=bb_mask,
                sc_w1=sc_w1, sc_b1=sc_b1, sc_w2=sc_w2, sc_b2=sc_b2,
                sc_w3=sc_w3, sc_b3=sc_b3, scmv=scmv, sc_const=sc_const,
                om_aff=om_aff)


_C = _build_constants()
_BB_MASK = _C['bb_mask']


def _block_body(bt_ref, x_ref, scmv_ref, tab_ref,
                bbw1_ref, bbb1_ref, bbw2_ref, bbb2_ref, bbw3_ref, bbb3_ref,
                scw1_ref, scb1_ref, scw2_ref, scb2_ref, scw3_ref, scb3_ref,
                out_ref):
    x = x_ref[...]
    scmv = scmv_ref[0]
    tab = tab_ref[0, 0]

    z0 = x[:, 0:1]
    z1 = x[:, 1:2]
    xom = x[:, 2]
    valid = scmv[6]
    zsc = x[:, 3:8] * valid[None, :]

    ld_bb = jnp.zeros((x.shape[0], 1), jnp.float32)
    ld_sc = jnp.zeros(x.shape[0], jnp.float32)

    def mlp(zm, wref, bref1, w2ref, bref2, w3ref, bref3, idx, final_tanh):
        h = jnp.tanh(jnp.dot(zm, wref[0, idx],
                             preferred_element_type=jnp.float32)
                     + bref1[0, idx][None, :])
        h = jnp.tanh(jnp.dot(h.astype(jnp.bfloat16), w2ref[0, idx],
                             preferred_element_type=jnp.float32)
                     + bref2[0, idx][None, :])
        o = jnp.dot(h, w3ref[0, idx],
                    preferred_element_type=jnp.float32) + bref3[0, idx][None, :]
        return jnp.tanh(o) if final_tanh else o

    def mlp_bb(u, a, b, idx, final_tanh):
        h = jnp.tanh(u * bbw1_ref[0, idx, a][None, :] + bbb1_ref[0, idx][None, :])
        h = jnp.tanh(jnp.dot(h.astype(jnp.bfloat16), bbw2_ref[0, idx],
                             preferred_element_type=jnp.float32)
                     + bbb2_ref[0, idx][None, :])
        o = jnp.dot(h, bbw3_ref[0, idx, :, b:b + 1],
                    preferred_element_type=jnp.float32) + bbb3_ref[0, idx, b]
        return jnp.tanh(o) if final_tanh else o

    for k in range(_NC):
        a = int(_BB_MASK[k][1] > 0.5)
        b = 1 - a
        u = z1 if a == 1 else z0
        zi = z0 if a == 1 else z1
        s = mlp_bb(u, a, b, 2 * k, True)
        t = mlp_bb(u, a, b, 2 * k + 1, False)
        zi = (zi - t) * jnp.exp(-s)
        if a == 1:
            z0 = zi
        else:
            z1 = zi
        ld_bb = ld_bb - s

        msc = scmv[k]
        zm5 = zsc * msc[None, :]
        s5 = mlp(zm5, scw1_ref, scb1_ref, scw2_ref, scb2_ref, scw3_ref,
                 scb3_ref, 2 * k, True) * (1.0 - msc)[None, :]
        t5 = mlp(zm5, scw1_ref, scb1_ref, scw2_ref, scb2_ref, scw3_ref,
                 scb3_ref, 2 * k + 1, False) * (1.0 - msc)[None, :]
        zsc = zm5 + (1.0 - msc)[None, :] * (zsc - t5) * jnp.exp(-s5)
        ld_sc = ld_sc - jnp.sum(s5, axis=-1)

    bb_logp = (-0.5 * (z0 * z0 + z1 * z1 + 2.0 * _LOG2PI) + ld_bb)[:, 0]
    sc_logp = (-0.5 * jnp.sum(zsc * zsc, axis=-1) - 0.5 * _LOG2PI * _SCF
               + tab[4] + ld_sc)
    zo = tab[0] * xom + tab[1]
    om_logp = -0.5 * (zo * zo + _LOG2PI) + tab[2]

    bb_p = jnp.minimum(bb_logp * tab[5], 5.0)
    om_p = om_logp * tab[6]
    sc_p = jnp.minimum(sc_logp * tab[3], 5.0)
    vals = jnp.clip(-(bb_p + om_p + sc_p), 0.0, 5.0)
    out_ref[...] = vals.reshape(out_ref.shape)


def kernel(atom_description, angles, alternatives, weightBB, weightOmega,
           weightSC):
    B, Cn, R, A, F = angles.shape
    n_res = B * Cn * R
    apr = atom_description.shape[0] // n_res
    resname = atom_description.reshape(n_res, apr,
                                       atom_description.shape[1])[:, 0, 3]
    resname = resname.astype(jnp.int32)

    nblk = n_res // _BLK + _NTYPES
    npad = nblk * _BLK

    order = jnp.argsort(resname).astype(jnp.int32)
    counts = jnp.bincount(resname, length=_NTYPES).astype(jnp.int32)
    padded = ((counts + _BLK - 1) // _BLK) * _BLK
    pstart = jnp.concatenate([jnp.zeros((1,), jnp.int32),
                              jnp.cumsum(padded)[:-1].astype(jnp.int32)])
    start = jnp.concatenate([jnp.zeros((1,), jnp.int32),
                             jnp.cumsum(counts)[:-1].astype(jnp.int32)])
    rs = resname[order]
    p = jnp.arange(n_res, dtype=jnp.int32)
    slot_sorted = pstart[rs] + (p - start[rs])
    slot_of_res = jnp.zeros((n_res,), jnp.int32).at[order].set(slot_sorted)
    res_of_slot = jnp.zeros((npad,), jnp.int32).at[slot_sorted].set(order)
    bt = jnp.zeros((nblk,), jnp.int32).at[slot_sorted // _BLK].set(rs)

    xg = angles.reshape(n_res, A * F)[res_of_slot].reshape(npad * A, F)

    mult_bb = 1.0 - jnp.tanh(-weightBB[0])
    mult_om = 1.0 - jnp.tanh(-weightOmega[0])
    mult_sc = 1.0 - jnp.tanh(-weightSC)
    tab = jnp.stack([
        jnp.asarray(_C['om_aff'][:, 0]),
        jnp.asarray(_C['om_aff'][:, 1]),
        jnp.asarray(_C['om_aff'][:, 2]),
        mult_sc,
        jnp.asarray(_C['sc_const']),
        jnp.full((_NTYPES,), mult_bb),
        jnp.full((_NTYPES,), mult_om),
        jnp.zeros((_NTYPES,)),
    ], axis=1).astype(jnp.float32).reshape(_NTYPES, 1, 8)

    def im_x(b, bt_r):
        return (b, 0)

    def im_t2(b, bt_r):
        return (bt_r[b], 0)

    def im_t3(b, bt_r):
        return (bt_r[b], 0, 0)

    def im_t4(b, bt_r):
        return (bt_r[b], 0, 0, 0)

    grid_spec = pltpu.PrefetchScalarGridSpec(
        num_scalar_prefetch=1,
        grid=(nblk,),
        in_specs=[
            pl.BlockSpec((_BLK * A, F), im_x),
            pl.BlockSpec((1, 8, _SCF), im_t3),
            pl.BlockSpec((1, 1, 8), im_t3),
            pl.BlockSpec((1, 2 * _NC, 2, _HIDDEN), im_t4),
            pl.BlockSpec((1, 2 * _NC, _HIDDEN), im_t3),
            pl.BlockSpec((1, 2 * _NC, _HIDDEN, _HIDDEN), im_t4),
            pl.BlockSpec((1, 2 * _NC, _HIDDEN), im_t3),
            pl.BlockSpec((1, 2 * _NC, _HIDDEN, 2), im_t4),
            pl.BlockSpec((1, 2 * _NC, 2), im_t3),
            pl.BlockSpec((1, 2 * _NC, _SCF, _HIDDEN), im_t4),
            pl.BlockSpec((1, 2 * _NC, _HIDDEN), im_t3),
            pl.BlockSpec((1, 2 * _NC, _HIDDEN, _HIDDEN), im_t4),
            pl.BlockSpec((1, 2 * _NC, _HIDDEN), im_t3),
            pl.BlockSpec((1, 2 * _NC, _HIDDEN, _SCF), im_t4),
            pl.BlockSpec((1, 2 * _NC, _SCF), im_t3),
        ],
        out_specs=pl.BlockSpec((_BLK, A), im_x),
    )

    out = pl.pallas_call(
        _block_body,
        grid_spec=grid_spec,
        out_shape=jax.ShapeDtypeStruct((npad, A), jnp.float32),
    )(bt, xg,
      jnp.asarray(_C['scmv']), tab,
      jnp.asarray(_C['bb_w1']), jnp.asarray(_C['bb_b1']),
      jnp.asarray(_C['bb_w2'], jnp.bfloat16), jnp.asarray(_C['bb_b2']),
      jnp.asarray(_C['bb_w3']), jnp.asarray(_C['bb_b3']),
      jnp.asarray(_C['sc_w1']), jnp.asarray(_C['sc_b1']),
      jnp.asarray(_C['sc_w2'], jnp.bfloat16), jnp.asarray(_C['sc_b2']),
      jnp.asarray(_C['sc_w3']), jnp.asarray(_C['sc_b3']))

    bbScore = out[slot_of_res].reshape(B, Cn, R, A)
    rotamerViolation = jnp.zeros_like(bbScore)
    return (bbScore, rotamerViolation)
